# Initial kernel scaffold; baseline (speedup 1.0000x reference)
import jax
import jax.numpy as jnp
from jax.experimental import pallas as pl


def kernel(x, sim_matrix, gates):
    raise NotImplementedError("write your pallas kernel here")


# speedup vs baseline: 12.6882x; 12.6882x over previous
import functools

import jax
import jax.numpy as jnp
import numpy as np
from jax.experimental import pallas as pl
from jax.experimental.pallas import tpu as pltpu

_NEG = -float(jnp.finfo(jnp.bfloat16).max)


def _sortable(v):
    b = jax.lax.bitcast_convert_type(v, jnp.int32)
    key = b ^ (jax.lax.shift_right_arithmetic(b, 31) & jnp.int32(0x7FFFFFFF))
    return key - jnp.int32(_KEY_LO)


_KEY_HI = int(np.array(1.01, np.float32).view(np.int32))
_KEY_LO = int(np.array(-1.01, np.float32).view(np.int32)
              ^ np.int32(0x7FFFFFFF))


def _body(x_ref, sim_ref, g_ref, probs_ref, pre_ref, mask_ref):
    x = x_ref[...]
    sim = sim_ref[...]
    B, H = x.shape
    E = sim.shape[1]
    k = E // 2

    ones_he = jnp.ones((H, E), jnp.float32)
    ones_ee = jnp.ones((E, E), jnp.float32)
    r_ = jax.lax.broadcasted_iota(jnp.int32, (E, E), 0)
    c_ = jax.lax.broadcasted_iota(jnp.int32, (E, E), 1)
    ltri = jnp.where(r_ < c_, 1.0, 0.0)

    rn = jnp.sqrt(jnp.sum(x * x, axis=1, keepdims=True))
    xn = x / jnp.maximum(rn, 1e-12)
    cn = jnp.sqrt(jnp.sum(sim * sim, axis=0, keepdims=True))
    simn = sim / jnp.maximum(cn, 1e-12)

    logits = jnp.dot(xn, simn, preferred_element_type=jnp.float32,
                     precision=jax.lax.Precision.HIGHEST)

    thr = jax.nn.sigmoid(g_ref[...])
    pre = logits - thr
    gated = jnp.maximum(pre, 0.0)
    act_f = jnp.where(gated > 0.0, 1.0, 0.0)
    n_act = jnp.dot(act_f, ones_ee, preferred_element_type=jnp.float32)

    keys = _sortable(logits)
    lo = jnp.zeros((B, E), jnp.int32)
    hi = jnp.full((B, E), _KEY_HI - _KEY_LO, jnp.int32)
    kf = jnp.float32(k)
    for _ in range(31):
        mid = lo + jax.lax.shift_right_arithmetic(hi - lo, 1)
        gt_f = jnp.where(keys > mid, 1.0, 0.0)
        cnt = jnp.dot(gt_f, ones_ee, preferred_element_type=jnp.float32)
        p_ = cnt >= kf
        lo = jnp.where(p_, mid, lo)
        hi = jnp.where(p_, hi, mid)
    t = hi

    gt_f = jnp.where(keys > t, 1.0, 0.0)
    c_gt = jnp.dot(gt_f, ones_ee, preferred_element_type=jnp.float32)
    eq_f = jnp.where(keys == t, 1.0, 0.0)
    eq_before = jnp.dot(eq_f, ltri, preferred_element_type=jnp.float32)
    tie_keep = jnp.where(eq_before < kf - c_gt, 1.0, 0.0)
    fallback_f = gt_f + eq_f * tie_keep

    keep_f = jnp.where(n_act == 0.0, fallback_f, act_f)
    masked = jnp.where(keep_f > 0.0, gated, _NEG)
    e = jnp.exp(masked)
    s = jnp.dot(e, ones_ee, preferred_element_type=jnp.float32,
                precision=jax.lax.Precision.HIGHEST)
    p = e / s

    probs_ref[...] = p
    pre_ref[...] = pre
    mask_ref[...] = keep_f


@functools.partial(jax.jit, static_argnames=("block",))
def _run(x, sim_matrix, gates2, block=512):
    n, h = x.shape
    e = sim_matrix.shape[1]
    block = min(block, n)
    grid = n // block
    outs = pl.pallas_call(
        _body,
        grid=(grid,),
        in_specs=[
            pl.BlockSpec((block, h), lambda i: (i, 0)),
            pl.BlockSpec((h, e), lambda i: (0, 0)),
            pl.BlockSpec((1, e), lambda i: (0, 0)),
        ],
        out_specs=[pl.BlockSpec((block, e), lambda i: (i, 0))] * 3,
        out_shape=[jax.ShapeDtypeStruct((n, e), jnp.float32)] * 3,
        compiler_params=pltpu.CompilerParams(
            dimension_semantics=("arbitrary",),
        ),
    )(x, sim_matrix, gates2)
    return tuple(outs)


def kernel(x, sim_matrix, gates):
    gates2 = gates.reshape(1, gates.shape[0])
    return _run(x, sim_matrix, gates2)
